# baseline (speedup 1.0000x reference)
import functools

import jax
import jax.numpy as jnp
from jax import lax
from jax.experimental import pallas as pl
from jax.experimental.pallas import tpu as pltpu
from jax.experimental.pallas import tpu_sc as plsc

N = 10000
D = 128
E = 320000
EPS = 1e-5

NC = 2
NS = 16
NW = NC * NS

CH = 128
EW = 80
EW0 = 152
EW1 = 8
BB = 4
E_PAD = NW * EW * CH
N_ACC = 10008
ZR = 1000


_mesh = functools.partial(
    plsc.VectorSubcoreMesh, core_axis_name="c", subcore_axis_name="s",
    num_cores=NC, num_subcores=NS)


def _worker_id():
    c = lax.axis_index("c")
    s = lax.axis_index("s")
    return c, s, c * NS + s


def _zero_acc(zeros_hbm, acc_sh, s):
    @pl.when(s < 10)
    def _():
        pltpu.sync_copy(zeros_hbm, acc_sh.at[pl.ds(s * ZR, ZR)])


def _copy_out(acc_sh, out_hbm, c, s):
    @pl.when(s < 10)
    def _():
        pltpu.sync_copy(acc_sh.at[pl.ds(s * ZR, ZR)],
                        out_hbm.at[c, pl.ds(s * ZR, ZR)])


def _sc_degrees_body(dst_hbm, zeros_hbm, ones_hbm, out_hbm,
                     idx_v, ones_v, sem, acc_sh):
    c, s, w = _worker_id()
    _zero_acc(zeros_hbm, acc_sh, s)
    pltpu.sync_copy(ones_hbm, ones_v)
    pltpu.sync_copy(dst_hbm.at[pl.ds(w * EW, EW)], idx_v)
    plsc.subcore_barrier()

    def step(j, carry):
        pltpu.sync_copy(ones_v, acc_sh.at[idx_v.at[j]], add=True)
        return carry

    lax.fori_loop(0, EW, step, 0)
    plsc.subcore_barrier()
    _copy_out(acc_sh, out_hbm, c, s)


def _sc_segment_sum_body(z2_hbm, sd_hbm, zeros_hbm, out_hbm,
                         sd_v, r0, r1, r2, g0, g1, g2, s0, s1, s2, acc_sh):
    c, s, w = _worker_id()
    _zero_acc(zeros_hbm, acc_sh, s)
    plsc.subcore_barrier()

    bufs = (r0, r1, r2)
    gsems = (g0, g1, g2)
    ssems = (s0, s1, s2)

    def gather(k, i):
        pltpu.async_copy(z2_hbm.at[sd_v.at[2 * k]], bufs[i], gsems[i])

    def wait_g(i):
        pltpu.make_async_copy(z2_hbm.at[sd_v.at[0]], bufs[i], gsems[i]).wait()

    def scatter(k, i):
        pltpu.async_copy(bufs[i], acc_sh.at[sd_v.at[2 * k + 1]], ssems[i],
                         add=True)

    def wait_s(i):
        pltpu.make_async_copy(bufs[i], acc_sh.at[sd_v.at[1]], ssems[i]).wait()

    base_w = jnp.where(c == 0, s * EW0, NS * EW0 + s * EW1)
    nb = jnp.where(c == 0, EW0 // BB, EW1 // BB)

    def block(bi, carry):
        pltpu.sync_copy(sd_hbm.at[pl.ds((base_w + bi * BB) * 2, 2 * BB)], sd_v)
        gather(0, 0)
        gather(1, 1)
        wait_g(0)
        scatter(0, 0)
        gather(2, 2)
        wait_g(1)
        scatter(1, 1)
        wait_s(0)
        gather(3, 0)
        wait_g(2)
        scatter(2, 2)
        wait_g(0)
        scatter(3, 0)
        wait_s(1)
        wait_s(2)
        wait_s(0)
        return carry

    lax.fori_loop(0, nb, block, 0)
    plsc.subcore_barrier()
    _copy_out(acc_sh, out_hbm, c, s)


def _deg_norm(deg_parts):
    deg = jnp.sum(deg_parts[0] + deg_parts[1], axis=-1) * (1.0 / 128.0)
    deg = jnp.maximum(deg, 1.0)
    return lax.rsqrt(deg)


def _tc_matmul_body(x_ref, w_ref, out_ref):
    out_ref[...] = jnp.dot(x_ref[...], w_ref[...],
                           precision=lax.Precision.HIGHEST,
                           preferred_element_type=jnp.float32)


def _tc_scale_body(z_ref, degp_ref, out_ref):
    norm = _deg_norm(degp_ref[...])
    out_ref[...] = z_ref[...] * norm[:, None]


def _tc_epilogue_body(aggp_ref, degp_ref, b_ref, gamma_ref, beta_ref, x_ref,
                      out_ref):
    norm = _deg_norm(degp_ref[...])
    h = (aggp_ref[0] + aggp_ref[1] + b_ref[...]) * norm[:, None]
    mean = jnp.mean(h, axis=0, keepdims=True)
    var = jnp.mean((h - mean) ** 2, axis=0, keepdims=True)
    h = (h - mean) * lax.rsqrt(var + EPS) * gamma_ref[...] + beta_ref[...]
    out_ref[...] = x_ref[...] + jnp.maximum(h, 0.0)


def kernel(features, edge_index, W, b, gamma, beta):
    src = edge_index[0]
    dst = edge_index[1]
    pad = E_PAD - E
    src_p = jnp.concatenate([src, jnp.zeros((pad,), jnp.int32)]).reshape(-1, CH)
    dst_p = jnp.concatenate([dst, jnp.full((pad,), N, jnp.int32)]).reshape(-1, CH)
    sd = jnp.stack([src_p, dst_p], axis=1).reshape(-1, CH)

    ones_rows = jnp.ones((CH, D), jnp.float32)
    zeros_blk = jnp.zeros((ZR, D), jnp.float32)

    deg_parts = pl.kernel(
        _sc_degrees_body,
        out_type=jax.ShapeDtypeStruct((NC, N, D), jnp.float32),
        mesh=_mesh(),
        scratch_types=[
            pltpu.VMEM((EW, CH), jnp.int32),
            pltpu.VMEM((CH, D), jnp.float32),
            pltpu.SemaphoreType.DMA,
            pltpu.VMEM_SHARED((N_ACC, D), jnp.float32),
        ],
    )(dst_p, zeros_blk, ones_rows)

    blk = 1000
    z = pl.pallas_call(
        _tc_matmul_body,
        grid=(N // blk,),
        in_specs=[
            pl.BlockSpec((blk, D), lambda i: (i, 0)),
            pl.BlockSpec((D, D), lambda i: (0, 0)),
        ],
        out_specs=pl.BlockSpec((blk, D), lambda i: (i, 0)),
        out_shape=jax.ShapeDtypeStruct((N, D), jnp.float32),
    )(features, W)
    z2 = pl.pallas_call(
        _tc_scale_body,
        grid=(N // blk,),
        in_specs=[
            pl.BlockSpec((blk, D), lambda i: (i, 0)),
            pl.BlockSpec((NC, blk, D), lambda i: (0, i, 0)),
        ],
        out_specs=pl.BlockSpec((blk, D), lambda i: (i, 0)),
        out_shape=jax.ShapeDtypeStruct((N, D), jnp.float32),
    )(z, deg_parts)

    agg_parts = pl.kernel(
        _sc_segment_sum_body,
        out_type=jax.ShapeDtypeStruct((NC, N, D), jnp.float32),
        mesh=_mesh(),
        scratch_types=[
            pltpu.VMEM((2 * BB, CH), jnp.int32),
            pltpu.VMEM((CH, D), jnp.float32),
            pltpu.VMEM((CH, D), jnp.float32),
            pltpu.VMEM((CH, D), jnp.float32),
            pltpu.SemaphoreType.DMA,
            pltpu.SemaphoreType.DMA,
            pltpu.SemaphoreType.DMA,
            pltpu.SemaphoreType.DMA,
            pltpu.SemaphoreType.DMA,
            pltpu.SemaphoreType.DMA,
            pltpu.VMEM_SHARED((N_ACC, D), jnp.float32),
        ],
    )(z2, sd, zeros_blk)

    out = pl.pallas_call(
        _tc_epilogue_body,
        out_shape=jax.ShapeDtypeStruct((N, D), jnp.float32),
    )(agg_parts, deg_parts, b.reshape(1, D), gamma.reshape(1, D),
      beta.reshape(1, D), features)
    return out


# speedup vs baseline: 1.0005x; 1.0005x over previous
import functools

import jax
import jax.numpy as jnp
from jax import lax
from jax.experimental import pallas as pl
from jax.experimental.pallas import tpu as pltpu
from jax.experimental.pallas import tpu_sc as plsc

N = 10000
D = 128
E = 320000
EPS = 1e-5

NC = 2
NS = 16
NW = NC * NS

CH = 128
EW = 80
EW0 = 148
EW1 = 12
BB = 4
E_PAD = NW * EW * CH
N_ACC = 10008
ZR = 1000


_mesh = functools.partial(
    plsc.VectorSubcoreMesh, core_axis_name="c", subcore_axis_name="s",
    num_cores=NC, num_subcores=NS)


def _worker_id():
    c = lax.axis_index("c")
    s = lax.axis_index("s")
    return c, s, c * NS + s


def _zero_acc(zeros_hbm, acc_sh, s):
    @pl.when(s < 10)
    def _():
        pltpu.sync_copy(zeros_hbm, acc_sh.at[pl.ds(s * ZR, ZR)])


def _copy_out(acc_sh, out_hbm, c, s):
    @pl.when(s < 10)
    def _():
        pltpu.sync_copy(acc_sh.at[pl.ds(s * ZR, ZR)],
                        out_hbm.at[c, pl.ds(s * ZR, ZR)])


def _sc_degrees_body(dst_hbm, zeros_hbm, ones_hbm, out_hbm,
                     idx_v, ones_v, sem, acc_sh):
    c, s, w = _worker_id()
    _zero_acc(zeros_hbm, acc_sh, s)
    pltpu.sync_copy(ones_hbm, ones_v)
    pltpu.sync_copy(dst_hbm.at[pl.ds(w * EW, EW)], idx_v)
    plsc.subcore_barrier()

    def step(j, carry):
        pltpu.sync_copy(ones_v, acc_sh.at[idx_v.at[j]], add=True)
        return carry

    lax.fori_loop(0, EW, step, 0)
    plsc.subcore_barrier()
    _copy_out(acc_sh, out_hbm, c, s)


def _sc_segment_sum_body(z2_hbm, sd_hbm, zeros_hbm, out_hbm,
                         sd_v, r0, r1, r2, g0, g1, g2, s0, s1, s2, acc_sh):
    c, s, w = _worker_id()
    _zero_acc(zeros_hbm, acc_sh, s)
    plsc.subcore_barrier()

    bufs = (r0, r1, r2)
    gsems = (g0, g1, g2)
    ssems = (s0, s1, s2)

    def gather(k, i):
        pltpu.async_copy(z2_hbm.at[sd_v.at[2 * k]], bufs[i], gsems[i])

    def wait_g(i):
        pltpu.make_async_copy(z2_hbm.at[sd_v.at[0]], bufs[i], gsems[i]).wait()

    def scatter(k, i):
        pltpu.async_copy(bufs[i], acc_sh.at[sd_v.at[2 * k + 1]], ssems[i],
                         add=True)

    def wait_s(i):
        pltpu.make_async_copy(bufs[i], acc_sh.at[sd_v.at[1]], ssems[i]).wait()

    base_w = jnp.where(c == 0, s * EW0, NS * EW0 + s * EW1)
    nb = jnp.where(c == 0, EW0 // BB, EW1 // BB)

    def block(bi, carry):
        pltpu.sync_copy(sd_hbm.at[pl.ds((base_w + bi * BB) * 2, 2 * BB)], sd_v)
        gather(0, 0)
        gather(1, 1)
        wait_g(0)
        scatter(0, 0)
        gather(2, 2)
        wait_g(1)
        scatter(1, 1)
        wait_s(0)
        gather(3, 0)
        wait_g(2)
        scatter(2, 2)
        wait_g(0)
        scatter(3, 0)
        wait_s(1)
        wait_s(2)
        wait_s(0)
        return carry

    lax.fori_loop(0, nb, block, 0)
    plsc.subcore_barrier()
    _copy_out(acc_sh, out_hbm, c, s)


def _deg_norm(deg_parts):
    deg = jnp.sum(deg_parts[0] + deg_parts[1], axis=-1) * (1.0 / 128.0)
    deg = jnp.maximum(deg, 1.0)
    return lax.rsqrt(deg)


def _tc_matmul_body(x_ref, w_ref, out_ref):
    out_ref[...] = jnp.dot(x_ref[...], w_ref[...],
                           precision=lax.Precision.HIGHEST,
                           preferred_element_type=jnp.float32)


def _tc_scale_body(z_ref, degp_ref, out_ref):
    norm = _deg_norm(degp_ref[...])
    out_ref[...] = z_ref[...] * norm[:, None]


def _tc_epilogue_body(aggp_ref, degp_ref, b_ref, gamma_ref, beta_ref, x_ref,
                      out_ref):
    norm = _deg_norm(degp_ref[...])
    h = (aggp_ref[0] + aggp_ref[1] + b_ref[...]) * norm[:, None]
    mean = jnp.mean(h, axis=0, keepdims=True)
    var = jnp.mean((h - mean) ** 2, axis=0, keepdims=True)
    h = (h - mean) * lax.rsqrt(var + EPS) * gamma_ref[...] + beta_ref[...]
    out_ref[...] = x_ref[...] + jnp.maximum(h, 0.0)


def kernel(features, edge_index, W, b, gamma, beta):
    src = edge_index[0]
    dst = edge_index[1]
    pad = E_PAD - E
    src_p = jnp.concatenate([src, jnp.zeros((pad,), jnp.int32)]).reshape(-1, CH)
    dst_p = jnp.concatenate([dst, jnp.full((pad,), N, jnp.int32)]).reshape(-1, CH)
    sd = jnp.stack([src_p, dst_p], axis=1).reshape(-1, CH)

    ones_rows = jnp.ones((CH, D), jnp.float32)
    zeros_blk = jnp.zeros((ZR, D), jnp.float32)

    deg_parts = pl.kernel(
        _sc_degrees_body,
        out_type=jax.ShapeDtypeStruct((NC, N, D), jnp.float32),
        mesh=_mesh(),
        scratch_types=[
            pltpu.VMEM((EW, CH), jnp.int32),
            pltpu.VMEM((CH, D), jnp.float32),
            pltpu.SemaphoreType.DMA,
            pltpu.VMEM_SHARED((N_ACC, D), jnp.float32),
        ],
    )(dst_p, zeros_blk, ones_rows)

    blk = 1000
    z = pl.pallas_call(
        _tc_matmul_body,
        grid=(N // blk,),
        in_specs=[
            pl.BlockSpec((blk, D), lambda i: (i, 0)),
            pl.BlockSpec((D, D), lambda i: (0, 0)),
        ],
        out_specs=pl.BlockSpec((blk, D), lambda i: (i, 0)),
        out_shape=jax.ShapeDtypeStruct((N, D), jnp.float32),
    )(features, W)
    z2 = pl.pallas_call(
        _tc_scale_body,
        grid=(N // blk,),
        in_specs=[
            pl.BlockSpec((blk, D), lambda i: (i, 0)),
            pl.BlockSpec((NC, blk, D), lambda i: (0, i, 0)),
        ],
        out_specs=pl.BlockSpec((blk, D), lambda i: (i, 0)),
        out_shape=jax.ShapeDtypeStruct((N, D), jnp.float32),
    )(z, deg_parts)

    agg_parts = pl.kernel(
        _sc_segment_sum_body,
        out_type=jax.ShapeDtypeStruct((NC, N, D), jnp.float32),
        mesh=_mesh(),
        scratch_types=[
            pltpu.VMEM((2 * BB, CH), jnp.int32),
            pltpu.VMEM((CH, D), jnp.float32),
            pltpu.VMEM((CH, D), jnp.float32),
            pltpu.VMEM((CH, D), jnp.float32),
            pltpu.SemaphoreType.DMA,
            pltpu.SemaphoreType.DMA,
            pltpu.SemaphoreType.DMA,
            pltpu.SemaphoreType.DMA,
            pltpu.SemaphoreType.DMA,
            pltpu.SemaphoreType.DMA,
            pltpu.VMEM_SHARED((N_ACC, D), jnp.float32),
        ],
    )(z2, sd, zeros_blk)

    out = pl.pallas_call(
        _tc_epilogue_body,
        out_shape=jax.ShapeDtypeStruct((N, D), jnp.float32),
    )(agg_parts, deg_parts, b.reshape(1, D), gamma.reshape(1, D),
      beta.reshape(1, D), features)
    return out
